# baseline (speedup 1.0000x reference)
import jax
import jax.numpy as jnp
from jax import lax
from jax.experimental import pallas as pl
from jax.experimental.pallas import tpu as pltpu
from jax.experimental.pallas import tpu_sc as plsc

N = 10000
D = 128
R = 16
E = 320000
EPS = 1e-5

NR = N * R
TRASH = 512
EPAD = 327680
ROWS = EPAD // 128
EXTRA = EPAD - E

DC = 8
NCH = D // DC
HALF = NR // 2
ACC2 = HALF + TRASH
ZR2 = ACC2 // 16

CACC = NR + TRASH
CZR = CACC // 16

NSC = 2
NTILE = 16

_mesh = plsc.VectorSubcoreMesh(core_axis_name="c", subcore_axis_name="s")
_sc_params = pltpu.CompilerParams(use_tc_tiling_on_sc=False)


def _fill16(ref, value, dtype):
    for l in range(128 // 16):
        ref[pl.ds(l * 16, 16)] = jnp.full((16,), value, dtype)


def _count_body(dst_hbm, et_hbm, zeros1_hbm, seg_hbm, cnt_hbm,
                acc, dstv, etv, segv, onesv, zbuf):
    ci = lax.axis_index("c")
    si = lax.axis_index("s")

    _fill16(onesv, 1.0, jnp.float32)
    pltpu.sync_copy(zeros1_hbm, zbuf)
    pltpu.sync_copy(zbuf, acc.at[pl.ds(si * CZR, CZR)])
    plsc.subcore_barrier()

    iters = ROWS // NSC // NTILE
    base = (ci * (ROWS // NSC) + si * iters).astype(jnp.int32)

    def body(i, _):
        eb = (base + i) * 128
        pltpu.sync_copy(dst_hbm.at[pl.ds(eb, 128)], dstv)
        pltpu.sync_copy(et_hbm.at[pl.ds(eb, 128)], etv)
        for l in range(128 // 16):
            sl = pl.ds(l * 16, 16)
            segv[sl] = dstv[sl] * R + etv[sl]
        pltpu.sync_copy(segv, seg_hbm.at[pl.ds(eb, 128)])
        pltpu.sync_copy(onesv, acc.at[segv], add=True)
        return 0

    lax.fori_loop(0, iters, body, 0)
    plsc.subcore_barrier()
    pltpu.sync_copy(acc.at[pl.ds(si * (NR // NTILE), NR // NTILE)],
                    zbuf.at[pl.ds(0, NR // NTILE)])
    pltpu.sync_copy(zbuf.at[pl.ds(0, NR // NTILE)],
                    cnt_hbm.at[pl.ds(ci * NR + si * (NR // NTILE),
                                     NR // NTILE)])


_count = pl.kernel(
    _count_body,
    out_type=[
        jax.ShapeDtypeStruct((EPAD,), jnp.int32),
        jax.ShapeDtypeStruct((NSC * NR,), jnp.float32),
    ],
    mesh=_mesh,
    compiler_params=_sc_params,
    scratch_types=[
        pltpu.VMEM_SHARED((CACC,), jnp.float32),
        pltpu.VMEM((128,), jnp.int32),
        pltpu.VMEM((128,), jnp.int32),
        pltpu.VMEM((128,), jnp.int32),
        pltpu.VMEM((128,), jnp.float32),
        pltpu.VMEM((CZR,), jnp.float32),
    ],
)


def _segsum_body(hc_hbm, src_hbm, seg_hbm, zeros8_hbm, s_hbm,
                 acc, htab, srcv, segv, idxv, rowsv, zbuf):
    ci = lax.axis_index("c")
    si = lax.axis_index("s")

    iters = ROWS // NTILE
    ebase = (si * iters).astype(jnp.int32)
    lo = ci * HALF

    for cc in range(NCH):
        pltpu.sync_copy(zeros8_hbm, zbuf)
        pltpu.sync_copy(zbuf, acc.at[pl.ds(si * ZR2, ZR2)])
        @pl.when(si < 10)
        def _stage():
            pltpu.sync_copy(
                hc_hbm.at[pl.ds(cc * N + si * 1000, 1000)],
                zbuf.at[pl.ds(0, 1000)])
            pltpu.sync_copy(zbuf.at[pl.ds(0, 1000)],
                            htab.at[pl.ds(si * 1000, 1000)])
        plsc.subcore_barrier()

        def body(i, _):
            eb = (ebase + i) * 128
            pltpu.sync_copy(src_hbm.at[pl.ds(eb, 128)], srcv)
            pltpu.sync_copy(seg_hbm.at[pl.ds(eb, 128)], segv)
            for l in range(128 // 16):
                sl = pl.ds(l * 16, 16)
                sv = segv[sl]
                local = sv - lo
                valid = (local >= 0) & (local < HALF)
                divert = HALF + (sv & (TRASH - 1))
                idxv[sl] = jnp.where(valid, local, divert)
            pltpu.sync_copy(htab.at[srcv], rowsv)
            pltpu.sync_copy(rowsv, acc.at[idxv], add=True)
            return 0

        lax.fori_loop(0, iters, body, 0)
        plsc.subcore_barrier()
        pltpu.sync_copy(acc.at[pl.ds(si * (HALF // NTILE), HALF // NTILE)],
                        zbuf.at[pl.ds(0, HALF // NTILE)])
        pltpu.sync_copy(
            zbuf.at[pl.ds(0, HALF // NTILE)],
            s_hbm.at[cc, pl.ds(lo + si * (HALF // NTILE), HALF // NTILE), :])
        plsc.subcore_barrier()


_segsum = pl.kernel(
    _segsum_body,
    out_type=jax.ShapeDtypeStruct((NCH, NR, DC), jnp.float32),
    mesh=_mesh,
    compiler_params=_sc_params,
    scratch_types=[
        pltpu.VMEM_SHARED((ACC2, DC), jnp.float32),
        pltpu.VMEM_SHARED((N, DC), jnp.float32),
        pltpu.VMEM((128,), jnp.int32),
        pltpu.VMEM((128,), jnp.int32),
        pltpu.VMEM((128,), jnp.int32),
        pltpu.VMEM((128, DC), jnp.float32),
        pltpu.VMEM((ZR2, DC), jnp.float32),
    ],
)


NBLK = 10
BLK = N // NBLK


def _pre_body(s_ref, cnt_ref, b128_ref, h_ref, wp_ref, root_ref, bias_ref,
              pre_ref, st_ref):
    cnt = cnt_ref[0] + cnt_ref[1]
    inv = 1.0 / jnp.maximum(cnt, 1.0)
    inv128 = jnp.dot(inv, b128_ref[...],
                     preferred_element_type=jnp.float32)
    agg = jnp.dot(h_ref[...], root_ref[...],
                  preferred_element_type=jnp.float32)
    for c in range(NCH):
        agg += jnp.dot(s_ref[c] * inv128, wp_ref[c],
                       preferred_element_type=jnp.float32)
    pre = agg + bias_ref[...]
    pre_ref[...] = pre
    st_ref[0, 0, :] = pre.sum(axis=0)
    st_ref[0, 1, :] = (pre * pre).sum(axis=0)


def _pre(s3, cnt3, b128, h, wp, root, bias2d):
    return pl.pallas_call(
        _pre_body,
        grid=(NBLK,),
        in_specs=[
            pl.BlockSpec((NCH, BLK, R * DC), lambda i: (0, i, 0)),
            pl.BlockSpec((NSC, BLK, R), lambda i: (0, i, 0)),
            pl.BlockSpec((R, R * DC), lambda i: (0, 0)),
            pl.BlockSpec((BLK, D), lambda i: (i, 0)),
            pl.BlockSpec((NCH, R * DC, D), lambda i: (0, 0, 0)),
            pl.BlockSpec((D, D), lambda i: (0, 0)),
            pl.BlockSpec((1, D), lambda i: (0, 0)),
        ],
        out_specs=[
            pl.BlockSpec((BLK, D), lambda i: (i, 0)),
            pl.BlockSpec((1, 2, D), lambda i: (i, 0, 0)),
        ],
        out_shape=[
            jax.ShapeDtypeStruct((N, D), jnp.float32),
            jax.ShapeDtypeStruct((NBLK, 2, D), jnp.float32),
        ],
    )(s3, cnt3, b128, h, wp, root, bias2d)


def _bn_body(pre_ref, st_ref, g_ref, b_ref, h_ref):
    mu = st_ref[:, 0, :].sum(axis=0) * (1.0 / N)
    msq = st_ref[:, 1, :].sum(axis=0) * (1.0 / N)
    var = msq - mu * mu
    scale = lax.rsqrt(var + EPS) * g_ref[0]
    h_ref[...] = jnp.maximum((pre_ref[...] - mu) * scale + b_ref[0], 0.0)


def _bn(pre, st, gamma2d, beta2d):
    return pl.pallas_call(
        _bn_body,
        out_shape=jax.ShapeDtypeStruct((N, D), jnp.float32),
    )(pre, st, gamma2d, beta2d)


def _chunked(h):
    return h.reshape(N, NCH, DC).transpose(1, 0, 2).reshape(NCH * N, DC)


def kernel(x, edge_index, edge_type, emb, W1, root1, bias1, gamma1, beta1,
           W2, root2, bias2, gamma2, beta2):
    src = edge_index[0]
    dst = edge_index[1]
    pad = jnp.arange(EXTRA, dtype=jnp.int32)
    src1 = jnp.concatenate([src, pad % N])
    dst1 = jnp.concatenate([dst, jnp.full((EXTRA,), N, jnp.int32)])
    et1 = jnp.concatenate([edge_type, pad % TRASH])
    zeros1 = jnp.zeros((CZR,), jnp.float32)
    zeros8 = jnp.zeros((ZR2, DC), jnp.float32)

    seg1, cnt_parts = _count(dst1, et1, zeros1)
    cnt3 = cnt_parts.reshape(NSC, N, R)

    b128 = jnp.repeat(jnp.eye(R, dtype=jnp.float32), DC, axis=1)
    wp1 = W1.reshape(R, NCH, DC, D).transpose(1, 0, 2, 3).reshape(NCH, R * DC, D)
    wp2 = W2.reshape(R, NCH, DC, D).transpose(1, 0, 2, 3).reshape(NCH, R * DC, D)

    h0 = jnp.take(emb, x, axis=0)

    s1 = _segsum(_chunked(h0), src1, seg1, zeros8)
    pre1, st1 = _pre(s1.reshape(NCH, N, R * DC), cnt3, b128, h0, wp1, root1,
                     bias1.reshape(1, D))
    h1 = _bn(pre1, st1, gamma1.reshape(1, D), beta1.reshape(1, D))

    s2 = _segsum(_chunked(h1), src1, seg1, zeros8)
    pre2, st2 = _pre(s2.reshape(NCH, N, R * DC), cnt3, b128, h1, wp2, root2,
                     bias2.reshape(1, D))
    return _bn(pre2, st2, gamma2.reshape(1, D), beta2.reshape(1, D))


# speedup vs baseline: 2.1444x; 2.1444x over previous
import jax
import jax.numpy as jnp
from jax import lax
from jax.experimental import pallas as pl
from jax.experimental.pallas import tpu as pltpu
from jax.experimental.pallas import tpu_sc as plsc

N = 10000
D = 128
R = 16
E = 320000
EPS = 1e-5

NR = N * R
TRASH = 512
EPAD = 327680
ROWS = EPAD // 128
EXTRA = EPAD - E

DC = 8
NCH = D // DC
HALF = NR // 2
ACC2 = HALF + TRASH
ZR2 = ACC2 // 16

CACC = NR + TRASH
CZR = CACC // 16

NSC = 2
NTILE = 16

_mesh = plsc.VectorSubcoreMesh(core_axis_name="c", subcore_axis_name="s")
_sc_params = pltpu.CompilerParams(use_tc_tiling_on_sc=False)


def _fill16(ref, value, dtype):
    for l in range(128 // 16):
        ref[pl.ds(l * 16, 16)] = jnp.full((16,), value, dtype)


def _count_body(dst_hbm, et_hbm, zeros1_hbm, seg_hbm, cnt_hbm,
                acc, dstv, etv, segv, onesv, zbuf):
    ci = lax.axis_index("c")
    si = lax.axis_index("s")

    _fill16(onesv, 1.0, jnp.float32)
    pltpu.sync_copy(zeros1_hbm, zbuf)
    pltpu.sync_copy(zbuf, acc.at[pl.ds(si * CZR, CZR)])
    plsc.subcore_barrier()

    iters = ROWS // NSC // NTILE
    base = (ci * (ROWS // NSC) + si * iters).astype(jnp.int32)

    def body(i, _):
        eb = (base + i) * 128
        pltpu.sync_copy(dst_hbm.at[pl.ds(eb, 128)], dstv)
        pltpu.sync_copy(et_hbm.at[pl.ds(eb, 128)], etv)
        for l in range(128 // 16):
            sl = pl.ds(l * 16, 16)
            segv[sl] = dstv[sl] * R + etv[sl]
        pltpu.sync_copy(segv, seg_hbm.at[pl.ds(eb, 128)])
        pltpu.sync_copy(onesv, acc.at[segv], add=True)
        return 0

    lax.fori_loop(0, iters, body, 0)
    plsc.subcore_barrier()
    pltpu.sync_copy(acc.at[pl.ds(si * (NR // NTILE), NR // NTILE)],
                    zbuf.at[pl.ds(0, NR // NTILE)])
    pltpu.sync_copy(zbuf.at[pl.ds(0, NR // NTILE)],
                    cnt_hbm.at[pl.ds(ci * NR + si * (NR // NTILE),
                                     NR // NTILE)])


_count = pl.kernel(
    _count_body,
    out_type=[
        jax.ShapeDtypeStruct((EPAD,), jnp.int32),
        jax.ShapeDtypeStruct((NSC * NR,), jnp.float32),
    ],
    mesh=_mesh,
    compiler_params=_sc_params,
    scratch_types=[
        pltpu.VMEM_SHARED((CACC,), jnp.float32),
        pltpu.VMEM((128,), jnp.int32),
        pltpu.VMEM((128,), jnp.int32),
        pltpu.VMEM((128,), jnp.int32),
        pltpu.VMEM((128,), jnp.float32),
        pltpu.VMEM((CZR,), jnp.float32),
    ],
)


NBUF = 8
EIT = ROWS // NTILE
SB = 16384


def _segsum_body(hc_hbm, pk_hbm, zeros8_hbm, s_hbm,
                 acc, parr, srcv, idxv, rowsv, zbuf, gsem, ssem):
    ci = lax.axis_index("c")
    si = lax.axis_index("s")
    lo = ci * HALF

    pltpu.sync_copy(pk_hbm.at[ci, pl.ds(si * EIT, EIT)], parr)

    def unpack(j, b, coff):
        for l in range(128 // 16):
            sl = pl.ds(l * 16, 16)
            p = parr[j, sl]
            srcv[b][sl] = (p & (SB - 1)) + coff
            idxv[b][sl] = lax.shift_right_logical(p, 14)

    def g_start(j, b):
        pltpu.async_copy(hc_hbm.at[srcv[b]], rowsv[b], gsem[b])

    def g_wait(j, b):
        pltpu.make_async_copy(hc_hbm.at[srcv[b]], rowsv[b], gsem[b]).wait()

    def s_start(j, b):
        pltpu.async_copy(rowsv[b], acc.at[idxv[b]], ssem[b], add=True)

    def s_wait(j, b):
        pltpu.make_async_copy(rowsv[b], acc.at[idxv[b]], ssem[b]).wait()

    def chunk_body(cc, _):
        coff = cc * N
        pltpu.sync_copy(zeros8_hbm, zbuf)
        pltpu.sync_copy(zbuf.at[pl.ds(0, ZR2 // 2)],
                        acc.at[pl.ds(si * ZR2, ZR2 // 2)])
        pltpu.sync_copy(zbuf.at[pl.ds(0, ZR2 // 2)],
                        acc.at[pl.ds(si * ZR2 + ZR2 // 2, ZR2 // 2)])
        plsc.subcore_barrier()

        for b in range(NBUF):
            unpack(jnp.int32(b), b, coff)
            g_start(jnp.int32(b), b)

        def body(g, _):
            j0 = g * NBUF
            for b in range(NBUF):
                g_wait(j0 + b, b)
                s_start(j0 + b, b)
            for b in range(NBUF):
                s_wait(j0 + b, b)
            for b in range(NBUF):
                unpack(j0 + NBUF + b, b, coff)
                g_start(j0 + NBUF + b, b)
            return 0

        lax.fori_loop(0, EIT // NBUF - 1, body, 0)
        j0 = jnp.int32(EIT - NBUF)
        for b in range(NBUF):
            g_wait(j0 + b, b)
            s_start(j0 + b, b)
        for b in range(NBUF):
            s_wait(j0 + b, b)

        plsc.subcore_barrier()
        for q in range(2):
            qr = HALF // NTILE // 2
            pltpu.sync_copy(
                acc.at[pl.ds(si * (HALF // NTILE) + q * qr, qr)],
                zbuf.at[pl.ds(0, qr)])
            pltpu.sync_copy(
                zbuf.at[pl.ds(0, qr)],
                s_hbm.at[cc, pl.ds(lo + si * (HALF // NTILE) + q * qr, qr),
                         :])
        plsc.subcore_barrier()
        return 0

    lax.fori_loop(0, NCH, chunk_body, 0)


_segsum = pl.kernel(
    _segsum_body,
    out_type=jax.ShapeDtypeStruct((NCH, NR, DC), jnp.float32),
    mesh=_mesh,
    compiler_params=_sc_params,
    scratch_types=[
        pltpu.VMEM_SHARED((ACC2, DC), jnp.float32),
        pltpu.VMEM((EIT, 128), jnp.int32),
        [pltpu.VMEM((128,), jnp.int32) for _ in range(NBUF)],
        [pltpu.VMEM((128,), jnp.int32) for _ in range(NBUF)],
        [pltpu.VMEM((128, DC), jnp.float32) for _ in range(NBUF)],
        pltpu.VMEM((ZR2 // 2, DC), jnp.float32),
        [pltpu.SemaphoreType.DMA for _ in range(NBUF)],
        [pltpu.SemaphoreType.DMA for _ in range(NBUF)],
    ],
)


NBLK = 10
BLK = N // NBLK


def _pre_body(s_ref, cnt_ref, b128_ref, h_ref, wp_ref, root_ref, bias_ref,
              pre_ref, st_ref):
    cnt = cnt_ref[0] + cnt_ref[1]
    inv = 1.0 / jnp.maximum(cnt, 1.0)
    inv128 = jnp.dot(inv, b128_ref[...],
                     preferred_element_type=jnp.float32)
    agg = jnp.dot(h_ref[...], root_ref[...],
                  preferred_element_type=jnp.float32)
    for c in range(NCH):
        agg += jnp.dot(s_ref[c] * inv128, wp_ref[c],
                       preferred_element_type=jnp.float32)
    pre = agg + bias_ref[...]
    pre_ref[...] = pre
    st_ref[0, 0, :] = pre.sum(axis=0)
    st_ref[0, 1, :] = (pre * pre).sum(axis=0)


def _pre(s3, cnt3, b128, h, wp, root, bias2d):
    return pl.pallas_call(
        _pre_body,
        grid=(NBLK,),
        in_specs=[
            pl.BlockSpec((NCH, BLK, R * DC), lambda i: (0, i, 0)),
            pl.BlockSpec((NSC, BLK, R), lambda i: (0, i, 0)),
            pl.BlockSpec((R, R * DC), lambda i: (0, 0)),
            pl.BlockSpec((BLK, D), lambda i: (i, 0)),
            pl.BlockSpec((NCH, R * DC, D), lambda i: (0, 0, 0)),
            pl.BlockSpec((D, D), lambda i: (0, 0)),
            pl.BlockSpec((1, D), lambda i: (0, 0)),
        ],
        out_specs=[
            pl.BlockSpec((BLK, D), lambda i: (i, 0)),
            pl.BlockSpec((1, 2, D), lambda i: (i, 0, 0)),
        ],
        out_shape=[
            jax.ShapeDtypeStruct((N, D), jnp.float32),
            jax.ShapeDtypeStruct((NBLK, 2, D), jnp.float32),
        ],
    )(s3, cnt3, b128, h, wp, root, bias2d)


def _bn_body(pre_ref, st_ref, g_ref, b_ref, h_ref):
    mu = st_ref[:, 0, :].sum(axis=0) * (1.0 / N)
    msq = st_ref[:, 1, :].sum(axis=0) * (1.0 / N)
    var = msq - mu * mu
    scale = lax.rsqrt(var + EPS) * g_ref[0]
    h_ref[...] = jnp.maximum((pre_ref[...] - mu) * scale + b_ref[0], 0.0)


def _bn(pre, st, gamma2d, beta2d):
    return pl.pallas_call(
        _bn_body,
        out_shape=jax.ShapeDtypeStruct((N, D), jnp.float32),
    )(pre, st, gamma2d, beta2d)


def _chunked(h):
    return h.reshape(N, NCH, DC).transpose(1, 0, 2).reshape(NCH * N, DC)


def kernel(x, edge_index, edge_type, emb, W1, root1, bias1, gamma1, beta1,
           W2, root2, bias2, gamma2, beta2):
    src = edge_index[0]
    dst = edge_index[1]
    pad = jnp.arange(EXTRA, dtype=jnp.int32)
    src1 = jnp.concatenate([src, pad % N])
    dst1 = jnp.concatenate([dst, jnp.full((EXTRA,), N, jnp.int32)])
    et1 = jnp.concatenate([edge_type, pad % TRASH])
    zeros1 = jnp.zeros((CZR,), jnp.float32)
    zeros8 = jnp.zeros((ZR2 // 2, DC), jnp.float32)

    seg1, cnt_parts = _count(dst1, et1, zeros1)
    cnt3 = cnt_parts.reshape(NSC, N, R)

    b128 = jnp.repeat(jnp.eye(R, dtype=jnp.float32), DC, axis=1)
    wp1 = W1.reshape(R, NCH, DC, D).transpose(1, 0, 2, 3).reshape(NCH, R * DC, D)
    wp2 = W2.reshape(R, NCH, DC, D).transpose(1, 0, 2, 3).reshape(NCH, R * DC, D)

    h0 = jnp.take(emb, x, axis=0)

    halves = []
    for half in range(NSC):
        local = seg1 - half * HALF
        valid = (local >= 0) & (local < HALF)
        divert = HALF + (seg1 & (TRASH - 1))
        idx = jnp.where(valid, local, divert)
        halves.append(idx * SB + src1)
    packed = jnp.stack(halves).reshape(NSC, ROWS, 128)

    s1 = _segsum(_chunked(h0), packed, zeros8)
    pre1, st1 = _pre(s1.reshape(NCH, N, R * DC), cnt3, b128, h0, wp1, root1,
                     bias1.reshape(1, D))
    h1 = _bn(pre1, st1, gamma1.reshape(1, D), beta1.reshape(1, D))

    s2 = _segsum(_chunked(h1), packed, zeros8)
    pre2, st2 = _pre(s2.reshape(NCH, N, R * DC), cnt3, b128, h1, wp2, root2,
                     bias2.reshape(1, D))
    return _bn(pre2, st2, gamma2.reshape(1, D), beta2.reshape(1, D))


# speedup vs baseline: 2.1942x; 1.0232x over previous
import jax
import jax.numpy as jnp
from jax import lax
from jax.experimental import pallas as pl
from jax.experimental.pallas import tpu as pltpu
from jax.experimental.pallas import tpu_sc as plsc

N = 10000
D = 128
R = 16
E = 320000
EPS = 1e-5

NR = N * R
TRASH = 512
EPAD = 327680
ROWS = EPAD // 128
EXTRA = EPAD - E

DC = 8
NCH = D // DC
HALF = NR // 2
ACC2 = HALF + TRASH
ZR2 = ACC2 // 16

CACC = NR + TRASH
CZR = CACC // 16

NSC = 2
NTILE = 16

_mesh = plsc.VectorSubcoreMesh(core_axis_name="c", subcore_axis_name="s")
_sc_params = pltpu.CompilerParams(use_tc_tiling_on_sc=False)


def _fill16(ref, value, dtype):
    for l in range(128 // 16):
        ref[pl.ds(l * 16, 16)] = jnp.full((16,), value, dtype)


def _count_body(dst_hbm, et_hbm, zeros1_hbm, seg_hbm, cnt_hbm,
                acc, dstv, etv, segv, onesv, zbuf):
    ci = lax.axis_index("c")
    si = lax.axis_index("s")

    _fill16(onesv, 1.0, jnp.float32)
    pltpu.sync_copy(zeros1_hbm, zbuf)
    pltpu.sync_copy(zbuf, acc.at[pl.ds(si * CZR, CZR)])
    plsc.subcore_barrier()

    iters = ROWS // NSC // NTILE
    base = (ci * (ROWS // NSC) + si * iters).astype(jnp.int32)

    def body(i, _):
        eb = (base + i) * 128
        pltpu.sync_copy(dst_hbm.at[pl.ds(eb, 128)], dstv)
        pltpu.sync_copy(et_hbm.at[pl.ds(eb, 128)], etv)
        for l in range(128 // 16):
            sl = pl.ds(l * 16, 16)
            segv[sl] = dstv[sl] * R + etv[sl]
        pltpu.sync_copy(segv, seg_hbm.at[pl.ds(eb, 128)])
        pltpu.sync_copy(onesv, acc.at[segv], add=True)
        return 0

    lax.fori_loop(0, iters, body, 0)
    plsc.subcore_barrier()
    pltpu.sync_copy(acc.at[pl.ds(si * (NR // NTILE), NR // NTILE)],
                    zbuf.at[pl.ds(0, NR // NTILE)])
    pltpu.sync_copy(zbuf.at[pl.ds(0, NR // NTILE)],
                    cnt_hbm.at[pl.ds(ci * NR + si * (NR // NTILE),
                                     NR // NTILE)])


_count = pl.kernel(
    _count_body,
    out_type=[
        jax.ShapeDtypeStruct((EPAD,), jnp.int32),
        jax.ShapeDtypeStruct((NSC * NR,), jnp.float32),
    ],
    mesh=_mesh,
    compiler_params=_sc_params,
    scratch_types=[
        pltpu.VMEM_SHARED((CACC,), jnp.float32),
        pltpu.VMEM((128,), jnp.int32),
        pltpu.VMEM((128,), jnp.int32),
        pltpu.VMEM((128,), jnp.int32),
        pltpu.VMEM((128,), jnp.float32),
        pltpu.VMEM((CZR,), jnp.float32),
    ],
)


NBUF = 8
EIT = ROWS // NTILE
SB = 16384


def _segsum_body(hc_hbm, pk_hbm, zeros8_hbm, s_hbm,
                 acc, parr, srcv, idxv, rowsv, zbuf, gsem, ssem):
    ci = lax.axis_index("c")
    si = lax.axis_index("s")
    lo = ci * HALF

    pltpu.sync_copy(pk_hbm.at[ci, pl.ds(si * EIT, EIT)], parr)

    def unpack(j, b, coff):
        for l in range(128 // 16):
            sl = pl.ds(l * 16, 16)
            p = parr[j, sl]
            srcv[b][sl] = lax.shift_left(p & (SB - 1), 4) + coff
            idxv[b][sl] = lax.shift_right_logical(p, 14)

    def g_start(j, b):
        pltpu.async_copy(hc_hbm.at[srcv[b]], rowsv[b], gsem[b])

    def g_wait(j, b):
        pltpu.make_async_copy(hc_hbm.at[srcv[b]], rowsv[b], gsem[b]).wait()

    def s_start(j, b):
        pltpu.async_copy(rowsv[b], acc.at[idxv[b]], ssem[b], add=True)

    def s_wait(j, b):
        pltpu.make_async_copy(rowsv[b], acc.at[idxv[b]], ssem[b]).wait()

    def chunk_body(cc, _):
        coff = cc
        pltpu.sync_copy(zeros8_hbm, zbuf)
        pltpu.sync_copy(zbuf.at[pl.ds(0, ZR2 // 2)],
                        acc.at[pl.ds(si * ZR2, ZR2 // 2)])
        pltpu.sync_copy(zbuf.at[pl.ds(0, ZR2 // 2)],
                        acc.at[pl.ds(si * ZR2 + ZR2 // 2, ZR2 // 2)])
        plsc.subcore_barrier()

        for b in range(NBUF):
            unpack(jnp.int32(b), b, coff)
            g_start(jnp.int32(b), b)

        def body(g, _):
            j0 = g * NBUF
            for b in range(NBUF):
                g_wait(j0 + b, b)
                s_start(j0 + b, b)
            for b in range(NBUF):
                s_wait(j0 + b, b)
            for b in range(NBUF):
                unpack(j0 + NBUF + b, b, coff)
                g_start(j0 + NBUF + b, b)
            return 0

        lax.fori_loop(0, EIT // NBUF - 1, body, 0)
        j0 = jnp.int32(EIT - NBUF)
        for b in range(NBUF):
            g_wait(j0 + b, b)
            s_start(j0 + b, b)
        for b in range(NBUF):
            s_wait(j0 + b, b)

        plsc.subcore_barrier()
        for q in range(2):
            qr = HALF // NTILE // 2
            pltpu.sync_copy(
                acc.at[pl.ds(si * (HALF // NTILE) + q * qr, qr)],
                zbuf.at[pl.ds(0, qr)])
            pltpu.sync_copy(
                zbuf.at[pl.ds(0, qr)],
                s_hbm.at[cc, pl.ds(lo + si * (HALF // NTILE) + q * qr, qr),
                         :])
        plsc.subcore_barrier()
        return 0

    lax.fori_loop(0, NCH, chunk_body, 0)


_segsum = pl.kernel(
    _segsum_body,
    out_type=jax.ShapeDtypeStruct((NCH, NR, DC), jnp.float32),
    mesh=_mesh,
    compiler_params=_sc_params,
    scratch_types=[
        pltpu.VMEM_SHARED((ACC2, DC), jnp.float32),
        pltpu.VMEM((EIT, 128), jnp.int32),
        [pltpu.VMEM((128,), jnp.int32) for _ in range(NBUF)],
        [pltpu.VMEM((128,), jnp.int32) for _ in range(NBUF)],
        [pltpu.VMEM((128, DC), jnp.float32) for _ in range(NBUF)],
        pltpu.VMEM((ZR2 // 2, DC), jnp.float32),
        [pltpu.SemaphoreType.DMA for _ in range(NBUF)],
        [pltpu.SemaphoreType.DMA for _ in range(NBUF)],
    ],
)


NBLK = 10
BLK = N // NBLK


def _pre_body(s_ref, cnt_ref, b128_ref, h_ref, wp_ref, root_ref, bias_ref,
              pre_ref, st_ref):
    cnt = cnt_ref[0] + cnt_ref[1]
    inv = 1.0 / jnp.maximum(cnt, 1.0)
    inv128 = jnp.dot(inv, b128_ref[...],
                     preferred_element_type=jnp.float32)
    agg = jnp.dot(h_ref[...], root_ref[...],
                  preferred_element_type=jnp.float32)
    for c in range(NCH):
        agg += jnp.dot(s_ref[c] * inv128, wp_ref[c],
                       preferred_element_type=jnp.float32)
    pre = agg + bias_ref[...]
    pre_ref[...] = pre
    st_ref[0, 0, :] = pre.sum(axis=0)
    st_ref[0, 1, :] = (pre * pre).sum(axis=0)


def _pre(s3, cnt3, b128, h, wp, root, bias2d):
    return pl.pallas_call(
        _pre_body,
        grid=(NBLK,),
        in_specs=[
            pl.BlockSpec((NCH, BLK, R * DC), lambda i: (0, i, 0)),
            pl.BlockSpec((NSC, BLK, R), lambda i: (0, i, 0)),
            pl.BlockSpec((R, R * DC), lambda i: (0, 0)),
            pl.BlockSpec((BLK, D), lambda i: (i, 0)),
            pl.BlockSpec((NCH, R * DC, D), lambda i: (0, 0, 0)),
            pl.BlockSpec((D, D), lambda i: (0, 0)),
            pl.BlockSpec((1, D), lambda i: (0, 0)),
        ],
        out_specs=[
            pl.BlockSpec((BLK, D), lambda i: (i, 0)),
            pl.BlockSpec((1, 2, D), lambda i: (i, 0, 0)),
        ],
        out_shape=[
            jax.ShapeDtypeStruct((N, D), jnp.float32),
            jax.ShapeDtypeStruct((NBLK, 2, D), jnp.float32),
        ],
    )(s3, cnt3, b128, h, wp, root, bias2d)


def _bn_body(pre_ref, st_ref, g_ref, b_ref, h_ref):
    mu = st_ref[:, 0, :].sum(axis=0) * (1.0 / N)
    msq = st_ref[:, 1, :].sum(axis=0) * (1.0 / N)
    var = msq - mu * mu
    scale = lax.rsqrt(var + EPS) * g_ref[0]
    h_ref[...] = jnp.maximum((pre_ref[...] - mu) * scale + b_ref[0], 0.0)


def _bn(pre, st, gamma2d, beta2d):
    return pl.pallas_call(
        _bn_body,
        out_shape=jax.ShapeDtypeStruct((N, D), jnp.float32),
    )(pre, st, gamma2d, beta2d)


def kernel(x, edge_index, edge_type, emb, W1, root1, bias1, gamma1, beta1,
           W2, root2, bias2, gamma2, beta2):
    src = edge_index[0]
    dst = edge_index[1]
    pad = jnp.arange(EXTRA, dtype=jnp.int32)
    src1 = jnp.concatenate([src, pad % N])
    dst1 = jnp.concatenate([dst, jnp.full((EXTRA,), N, jnp.int32)])
    et1 = jnp.concatenate([edge_type, pad % TRASH])
    zeros1 = jnp.zeros((CZR,), jnp.float32)
    zeros8 = jnp.zeros((ZR2 // 2, DC), jnp.float32)

    seg1, cnt_parts = _count(dst1, et1, zeros1)
    cnt3 = cnt_parts.reshape(NSC, N, R)

    b128 = jnp.repeat(jnp.eye(R, dtype=jnp.float32), DC, axis=1)
    wp1 = W1.reshape(R, NCH, DC, D).transpose(1, 0, 2, 3).reshape(NCH, R * DC, D)
    wp2 = W2.reshape(R, NCH, DC, D).transpose(1, 0, 2, 3).reshape(NCH, R * DC, D)

    h0 = jnp.take(emb, x, axis=0)

    halves = []
    for half in range(NSC):
        local = seg1 - half * HALF
        valid = (local >= 0) & (local < HALF)
        divert = HALF + (seg1 & (TRASH - 1))
        idx = jnp.where(valid, local, divert)
        halves.append(idx * SB + src1)
    packed = jnp.stack(halves).reshape(NSC, ROWS, 128)

    s1 = _segsum(h0.reshape(N * NCH, DC), packed, zeros8)
    pre1, st1 = _pre(s1.reshape(NCH, N, R * DC), cnt3, b128, h0, wp1, root1,
                     bias1.reshape(1, D))
    h1 = _bn(pre1, st1, gamma1.reshape(1, D), beta1.reshape(1, D))

    s2 = _segsum(h1.reshape(N * NCH, DC), packed, zeros8)
    pre2, st2 = _pre(s2.reshape(NCH, N, R * DC), cnt3, b128, h1, wp2, root2,
                     bias2.reshape(1, D))
    return _bn(pre2, st2, gamma2.reshape(1, D), beta2.reshape(1, D))


# speedup vs baseline: 3.8256x; 1.7435x over previous
import jax
import jax.numpy as jnp
from jax import lax
from jax.experimental import pallas as pl
from jax.experimental.pallas import tpu as pltpu
from jax.experimental.pallas import tpu_sc as plsc

N = 10000
D = 128
R = 16
E = 320000
EPS = 1e-5

NR = N * R
TRASH = 512
EPAD = 327680
ROWS = EPAD // 128
EXTRA = EPAD - E

DC = 8
NCH = D // DC
HALF = NR // 2
ACC2 = HALF + TRASH
ZR2 = ACC2 // 16

CACC = NR + TRASH
CZR = CACC // 16

NSC = 2
NTILE = 16

_mesh = plsc.VectorSubcoreMesh(core_axis_name="c", subcore_axis_name="s")
_sc_params = pltpu.CompilerParams(use_tc_tiling_on_sc=False, needs_layout_passes=False)


def _fill16(ref, value, dtype):
    for l in range(128 // 16):
        ref[pl.ds(l * 16, 16)] = jnp.full((16,), value, dtype)


def _count_body(dst_hbm, et_hbm, zeros1_hbm, seg_hbm, cnt_hbm,
                acc, dstv, etv, segv, onesv, zbuf):
    ci = lax.axis_index("c")
    si = lax.axis_index("s")

    _fill16(onesv, 1.0, jnp.float32)
    pltpu.sync_copy(zeros1_hbm, zbuf)
    pltpu.sync_copy(zbuf, acc.at[pl.ds(si * CZR, CZR)])
    plsc.subcore_barrier()

    iters = ROWS // NSC // NTILE
    base = (ci * (ROWS // NSC) + si * iters).astype(jnp.int32)

    def body(i, _):
        eb = (base + i) * 128
        pltpu.sync_copy(dst_hbm.at[pl.ds(eb, 128)], dstv)
        pltpu.sync_copy(et_hbm.at[pl.ds(eb, 128)], etv)
        for l in range(128 // 16):
            sl = pl.ds(l * 16, 16)
            segv[sl] = dstv[sl] * R + etv[sl]
        pltpu.sync_copy(segv, seg_hbm.at[pl.ds(eb, 128)])
        pltpu.sync_copy(onesv, acc.at[segv], add=True)
        return 0

    lax.fori_loop(0, iters, body, 0)
    plsc.subcore_barrier()
    pltpu.sync_copy(acc.at[pl.ds(si * (NR // NTILE), NR // NTILE)],
                    zbuf.at[pl.ds(0, NR // NTILE)])
    pltpu.sync_copy(zbuf.at[pl.ds(0, NR // NTILE)],
                    cnt_hbm.at[pl.ds(ci * NR + si * (NR // NTILE),
                                     NR // NTILE)])


_count = pl.kernel(
    _count_body,
    out_type=[
        jax.ShapeDtypeStruct((EPAD,), jnp.int32),
        jax.ShapeDtypeStruct((NSC * NR,), jnp.float32),
    ],
    mesh=_mesh,
    compiler_params=_sc_params,
    scratch_types=[
        pltpu.VMEM_SHARED((CACC,), jnp.float32),
        pltpu.VMEM((128,), jnp.int32),
        pltpu.VMEM((128,), jnp.int32),
        pltpu.VMEM((128,), jnp.int32),
        pltpu.VMEM((128,), jnp.float32),
        pltpu.VMEM((CZR,), jnp.float32),
    ],
)


NBUF = 3
SB = 16384
HALFN = N // 2
NBK = 2 * R
BW = 64
CAP = E + NBK * BW
ACCD = 5120
AZR = ACCD // 16


def _segsum_body(h_hbm, pk_hbm, roff_hbm, s_hbm,
                 acc, roffv, pkv, srcv, idxv, rowsv, sbuf,
                 pksem, gsem, ssem):
    ci = lax.axis_index("c")
    si = lax.axis_index("s")

    pltpu.sync_copy(roff_hbm, roffv)

    def rdoff(b):
        tot = jnp.int32(0)
        for g in range(3):
            v = roffv[pl.ds(g * 16, 16)]
            idx = lax.iota(jnp.int32, 16) + g * 16
            tot = tot + jnp.sum(jnp.where(idx == b, v, 0))
        return tot

    def pk_start(j, b, start):
        row = (start + 16 * j) * BW
        pltpu.async_copy(pk_hbm.at[pl.ds(row, BW)], pkv[b], pksem[b])

    def pk_wait(j, b, start):
        row = (start + 16 * j) * BW
        pltpu.make_async_copy(pk_hbm.at[pl.ds(row, BW)], pkv[b],
                              pksem[b]).wait()

    def unpack(b):
        for l in range(BW // 16):
            sl = pl.ds(l * 16, 16)
            p = pkv[b][sl]
            srcv[b][sl] = p & (SB - 1)
            idxv[b][sl] = lax.shift_right_logical(p, 14)

    def g_start(b):
        pltpu.async_copy(h_hbm.at[srcv[b]], rowsv[b], gsem[b])

    def g_wait(b):
        pltpu.make_async_copy(h_hbm.at[srcv[b]], rowsv[b], gsem[b]).wait()

    def s_start(b):
        pltpu.async_copy(rowsv[b], acc.at[idxv[b]], ssem[b], add=True)

    def s_wait(b):
        pltpu.make_async_copy(rowsv[b], acc.at[idxv[b]], ssem[b]).wait()

    def rel_body(r, _):
        bk = 2 * r + ci
        rlo = rdoff(bk)
        rhi = rdoff(bk + 1)
        start = rlo + si
        trips = jnp.maximum(-((start - rhi) // 16), 0)

        for i in range(BW):
            for l in range(128 // 16):
                rowsv[0][i, pl.ds(l * 16, 16)] = jnp.zeros((16,), jnp.float32)
        for q in range(AZR // BW):
            pltpu.sync_copy(rowsv[0], acc.at[pl.ds(si * AZR + q * BW, BW)])

        for b in range(NBUF):
            @pl.when(b < trips)
            def _():
                pk_start(jnp.int32(b), b, start)
        plsc.subcore_barrier()

        def grp(g, _):
            j0 = g * NBUF
            for b in range(NBUF):
                j = j0 + b

                @pl.when(j < trips)
                def _():
                    pk_wait(j, b, start)
                    unpack(b)
                    g_start(b)
            for b in range(NBUF):
                j = j0 + b

                @pl.when(j < trips)
                def _():
                    g_wait(b)
                    s_start(b)
            for b in range(NBUF):
                j = j0 + b

                @pl.when(j < trips)
                def _():
                    s_wait(b)

                @pl.when(j + NBUF < trips)
                def _():
                    pk_start(j + NBUF, b, start)
            return 0

        lax.fori_loop(0, -(-trips // NBUF), grp, 0)
        plsc.subcore_barrier()

        @pl.when(si < 10)
        def _spill():
            for q in range(5):
                pltpu.sync_copy(acc.at[pl.ds(si * 500 + q * 100, 100)], sbuf)
                pltpu.sync_copy(
                    sbuf,
                    s_hbm.at[r, pl.ds(ci * HALFN + si * 500 + q * 100, 100),
                             :])
        plsc.subcore_barrier()
        return 0

    lax.fori_loop(0, R, rel_body, 0)


_segsum = pl.kernel(
    _segsum_body,
    out_type=jax.ShapeDtypeStruct((R, N, D), jnp.float32),
    mesh=_mesh,
    compiler_params=_sc_params,
    scratch_types=[
        pltpu.VMEM_SHARED((ACCD, D), jnp.float32),
        pltpu.VMEM((48,), jnp.int32),
        [pltpu.VMEM((BW,), jnp.int32) for _ in range(NBUF)],
        [pltpu.VMEM((BW,), jnp.int32) for _ in range(NBUF)],
        [pltpu.VMEM((BW,), jnp.int32) for _ in range(NBUF)],
        [pltpu.VMEM((BW, D), jnp.float32) for _ in range(NBUF)],
        pltpu.VMEM((100, D), jnp.float32),
        [pltpu.SemaphoreType.DMA for _ in range(NBUF)],
        [pltpu.SemaphoreType.DMA for _ in range(NBUF)],
        [pltpu.SemaphoreType.DMA for _ in range(NBUF)],
    ],
)


NBLK = 10
BLK = N // NBLK


def _pre_body(s_ref, cnt_ref, h_ref, w_ref, root_ref, bias_ref,
              pre_ref, st_ref):
    cnt = cnt_ref[0] + cnt_ref[1]
    inv = 1.0 / jnp.maximum(cnt, 1.0)
    agg = jnp.dot(h_ref[...], root_ref[...],
                  preferred_element_type=jnp.float32)
    for r in range(R):
        agg += jnp.dot(s_ref[r] * inv[:, r][:, None], w_ref[r],
                       preferred_element_type=jnp.float32)
    pre = agg + bias_ref[...]
    pre_ref[...] = pre
    st_ref[0, 0, :] = pre.sum(axis=0)
    st_ref[0, 1, :] = (pre * pre).sum(axis=0)


def _pre(s3, cnt3, h, w, root, bias2d):
    return pl.pallas_call(
        _pre_body,
        grid=(NBLK,),
        in_specs=[
            pl.BlockSpec((R, BLK, D), lambda i: (0, i, 0)),
            pl.BlockSpec((NSC, BLK, R), lambda i: (0, i, 0)),
            pl.BlockSpec((BLK, D), lambda i: (i, 0)),
            pl.BlockSpec((R, D, D), lambda i: (0, 0, 0)),
            pl.BlockSpec((D, D), lambda i: (0, 0)),
            pl.BlockSpec((1, D), lambda i: (0, 0)),
        ],
        out_specs=[
            pl.BlockSpec((BLK, D), lambda i: (i, 0)),
            pl.BlockSpec((1, 2, D), lambda i: (i, 0, 0)),
        ],
        out_shape=[
            jax.ShapeDtypeStruct((N, D), jnp.float32),
            jax.ShapeDtypeStruct((NBLK, 2, D), jnp.float32),
        ],
    )(s3, cnt3, h, w, root, bias2d)


def _bn_body(pre_ref, st_ref, g_ref, b_ref, h_ref):
    mu = st_ref[:, 0, :].sum(axis=0) * (1.0 / N)
    msq = st_ref[:, 1, :].sum(axis=0) * (1.0 / N)
    var = msq - mu * mu
    scale = lax.rsqrt(var + EPS) * g_ref[0]
    h_ref[...] = jnp.maximum((pre_ref[...] - mu) * scale + b_ref[0], 0.0)


def _bn(pre, st, gamma2d, beta2d):
    return pl.pallas_call(
        _bn_body,
        out_shape=jax.ShapeDtypeStruct((N, D), jnp.float32),
    )(pre, st, gamma2d, beta2d)


def kernel(x, edge_index, edge_type, emb, W1, root1, bias1, gamma1, beta1,
           W2, root2, bias2, gamma2, beta2):
    src = edge_index[0]
    dst = edge_index[1]
    pad = jnp.arange(EXTRA, dtype=jnp.int32)
    src1 = jnp.concatenate([src, pad % N])
    dst1 = jnp.concatenate([dst, jnp.full((EXTRA,), N, jnp.int32)])
    et1 = jnp.concatenate([edge_type, pad % TRASH])
    zeros1 = jnp.zeros((CZR,), jnp.float32)

    seg1, cnt_parts = _count(dst1, et1, zeros1)
    cnt3 = cnt_parts.reshape(NSC, N, R)

    i32 = jnp.int32
    key = edge_type * 2 + (dst >= HALFN).astype(i32)
    sizes = jnp.bincount(key, length=NBK).astype(i32)
    psz = ((sizes + BW - 1) // BW) * BW
    z1 = jnp.zeros((1,), i32)
    poff = jnp.concatenate([z1, jnp.cumsum(psz, dtype=i32)])
    uoff = jnp.concatenate([z1, jnp.cumsum(sizes, dtype=i32)])
    order = jnp.argsort(key, stable=True)
    ks = key[order]
    rank = jnp.arange(E, dtype=i32) - uoff[ks]
    pos = poff[ks] + rank
    local = dst - (key & 1) * HALFN
    pv = local * SB + src
    ar = jnp.arange(CAP, dtype=i32)
    packed = (HALFN + (ar & (BW - 1))) * SB + (ar % 8192)
    packed = packed.at[pos].set(pv[order])
    roff = jnp.zeros((48,), i32).at[: NBK + 1].set(poff // BW)

    h0 = jnp.take(emb, x, axis=0)

    s1 = _segsum(h0, packed, roff)
    pre1, st1 = _pre(s1, cnt3, h0, W1, root1, bias1.reshape(1, D))
    h1 = _bn(pre1, st1, gamma1.reshape(1, D), beta1.reshape(1, D))

    s2 = _segsum(h1, packed, roff)
    pre2, st2 = _pre(s2, cnt3, h1, W2, root2, bias2.reshape(1, D))
    return _bn(pre2, st2, gamma2.reshape(1, D), beta2.reshape(1, D))


# speedup vs baseline: 3.9086x; 1.0217x over previous
import jax
import jax.numpy as jnp
from jax import lax
from jax.experimental import pallas as pl
from jax.experimental.pallas import tpu as pltpu
from jax.experimental.pallas import tpu_sc as plsc

N = 10000
D = 128
R = 16
E = 320000
EPS = 1e-5

NR = N * R
TRASH = 512
EPAD = 327680
ROWS = EPAD // 128
EXTRA = EPAD - E

DC = 8
NCH = D // DC
HALF = NR // 2
ACC2 = HALF + TRASH
ZR2 = ACC2 // 16

CACC = NR + TRASH
CZR = CACC // 16

NSC = 2
NTILE = 16

_mesh = plsc.VectorSubcoreMesh(core_axis_name="c", subcore_axis_name="s")
_sc_params = pltpu.CompilerParams(use_tc_tiling_on_sc=False, needs_layout_passes=False)


def _fill16(ref, value, dtype):
    for l in range(128 // 16):
        ref[pl.ds(l * 16, 16)] = jnp.full((16,), value, dtype)


def _count_body(dst_hbm, et_hbm, zeros1_hbm, seg_hbm, cnt_hbm,
                acc, dstv, etv, segv, onesv, zbuf):
    ci = lax.axis_index("c")
    si = lax.axis_index("s")

    _fill16(onesv, 1.0, jnp.float32)
    pltpu.sync_copy(zeros1_hbm, zbuf)
    pltpu.sync_copy(zbuf, acc.at[pl.ds(si * CZR, CZR)])
    plsc.subcore_barrier()

    iters = ROWS // NSC // NTILE
    base = (ci * (ROWS // NSC) + si * iters).astype(jnp.int32)

    def body(i, _):
        eb = (base + i) * 128
        pltpu.sync_copy(dst_hbm.at[pl.ds(eb, 128)], dstv)
        pltpu.sync_copy(et_hbm.at[pl.ds(eb, 128)], etv)
        for l in range(128 // 16):
            sl = pl.ds(l * 16, 16)
            segv[sl] = dstv[sl] * R + etv[sl]
        pltpu.sync_copy(segv, seg_hbm.at[pl.ds(eb, 128)])
        pltpu.sync_copy(onesv, acc.at[segv], add=True)
        return 0

    lax.fori_loop(0, iters, body, 0)
    plsc.subcore_barrier()
    pltpu.sync_copy(acc.at[pl.ds(si * (NR // NTILE), NR // NTILE)],
                    zbuf.at[pl.ds(0, NR // NTILE)])
    pltpu.sync_copy(zbuf.at[pl.ds(0, NR // NTILE)],
                    cnt_hbm.at[pl.ds(ci * NR + si * (NR // NTILE),
                                     NR // NTILE)])


_count = pl.kernel(
    _count_body,
    out_type=[
        jax.ShapeDtypeStruct((EPAD,), jnp.int32),
        jax.ShapeDtypeStruct((NSC * NR,), jnp.float32),
    ],
    mesh=_mesh,
    compiler_params=_sc_params,
    scratch_types=[
        pltpu.VMEM_SHARED((CACC,), jnp.float32),
        pltpu.VMEM((128,), jnp.int32),
        pltpu.VMEM((128,), jnp.int32),
        pltpu.VMEM((128,), jnp.int32),
        pltpu.VMEM((128,), jnp.float32),
        pltpu.VMEM((CZR,), jnp.float32),
    ],
)


NBUF = 3
SB = 16384
HALFN = N // 2
NBK = 2 * R
BW = 64
CAP = E + NBK * BW
ACCD = 5120
AZR = ACCD // 16


def _segsum_body(h_hbm, pk_hbm, roff_hbm, s_hbm,
                 acc, roffv, pkv, srcv, idxv, rowsv, sbuf,
                 pksem, gsem, ssem):
    ci = lax.axis_index("c")
    si = lax.axis_index("s")

    pltpu.sync_copy(roff_hbm, roffv)

    def rdoff(b):
        tot = jnp.int32(0)
        for g in range(3):
            v = roffv[pl.ds(g * 16, 16)]
            idx = lax.iota(jnp.int32, 16) + g * 16
            tot = tot + jnp.sum(jnp.where(idx == b, v, 0))
        return tot

    def pk_start(j, b, start):
        row = (start + 16 * j) * BW
        pltpu.async_copy(pk_hbm.at[pl.ds(row, BW)], pkv[b], pksem[b])

    def pk_wait(j, b, start):
        row = (start + 16 * j) * BW
        pltpu.make_async_copy(pk_hbm.at[pl.ds(row, BW)], pkv[b],
                              pksem[b]).wait()

    def unpack(b):
        for l in range(BW // 16):
            sl = pl.ds(l * 16, 16)
            p = pkv[b][sl]
            srcv[b][sl] = p & (SB - 1)
            idxv[b][sl] = lax.shift_right_logical(p, 14)

    def g_start(b):
        pltpu.async_copy(h_hbm.at[srcv[b]], rowsv[b], gsem[b])

    def g_wait(b):
        pltpu.make_async_copy(h_hbm.at[srcv[b]], rowsv[b], gsem[b]).wait()

    def s_start(b):
        pltpu.async_copy(rowsv[b], acc.at[idxv[b]], ssem[b], add=True)

    def s_wait(b):
        pltpu.make_async_copy(rowsv[b], acc.at[idxv[b]], ssem[b]).wait()

    def rel_body(r, _):
        bk = 2 * r + ci
        rlo = rdoff(bk)
        rhi = rdoff(bk + 1)
        start = rlo + si
        trips = jnp.maximum(-((start - rhi) // 16), 0)

        for i in range(BW):
            for l in range(128 // 16):
                rowsv[0][i, pl.ds(l * 16, 16)] = jnp.zeros((16,), jnp.float32)
        for q in range(AZR // BW):
            pltpu.sync_copy(rowsv[0], acc.at[pl.ds(si * AZR + q * BW, BW)])

        for b in range(NBUF):
            @pl.when(b < trips)
            def _():
                pk_start(jnp.int32(b), b, start)
        plsc.subcore_barrier()

        def grp(g, _):
            j0 = g * NBUF
            for b in range(NBUF):
                j = j0 + b

                @pl.when(j < trips)
                def _():
                    pk_wait(j, b, start)
                    unpack(b)
                    g_start(b)
            for b in range(NBUF):
                j = j0 + b

                @pl.when(j < trips)
                def _():
                    g_wait(b)
                    s_start(b)
            for b in range(NBUF):
                j = j0 + b

                @pl.when(j < trips)
                def _():
                    s_wait(b)

                @pl.when(j + NBUF < trips)
                def _():
                    pk_start(j + NBUF, b, start)
            return 0

        lax.fori_loop(0, -(-trips // NBUF), grp, 0)
        plsc.subcore_barrier()

        for q in range(3):
            ck = si + 16 * q

            @pl.when(ck < 40)
            def _spill():
                pltpu.sync_copy(acc.at[pl.ds(ck * 125, 125)], sbuf)
                pltpu.sync_copy(
                    sbuf, s_hbm.at[r, pl.ds(ci * HALFN + ck * 125, 125), :])
        plsc.subcore_barrier()
        return 0

    lax.fori_loop(0, R, rel_body, 0)


_segsum = pl.kernel(
    _segsum_body,
    out_type=jax.ShapeDtypeStruct((R, N, D), jnp.float32),
    mesh=_mesh,
    compiler_params=_sc_params,
    scratch_types=[
        pltpu.VMEM_SHARED((ACCD, D), jnp.float32),
        pltpu.VMEM((48,), jnp.int32),
        [pltpu.VMEM((BW,), jnp.int32) for _ in range(NBUF)],
        [pltpu.VMEM((BW,), jnp.int32) for _ in range(NBUF)],
        [pltpu.VMEM((BW,), jnp.int32) for _ in range(NBUF)],
        [pltpu.VMEM((BW, D), jnp.float32) for _ in range(NBUF)],
        pltpu.VMEM((125, D), jnp.float32),
        [pltpu.SemaphoreType.DMA for _ in range(NBUF)],
        [pltpu.SemaphoreType.DMA for _ in range(NBUF)],
        [pltpu.SemaphoreType.DMA for _ in range(NBUF)],
    ],
)


NBLK = 10
BLK = N // NBLK


def _pre_body(s_ref, cnt_ref, h_ref, w_ref, root_ref, bias_ref,
              pre_ref, st_ref):
    cnt = cnt_ref[0] + cnt_ref[1]
    inv = 1.0 / jnp.maximum(cnt, 1.0)
    agg = jnp.dot(h_ref[...], root_ref[...],
                  preferred_element_type=jnp.float32)
    for r in range(R):
        agg += jnp.dot(s_ref[r] * inv[:, r][:, None], w_ref[r],
                       preferred_element_type=jnp.float32)
    pre = agg + bias_ref[...]
    pre_ref[...] = pre
    st_ref[0, 0, :] = pre.sum(axis=0)
    st_ref[0, 1, :] = (pre * pre).sum(axis=0)


def _pre(s3, cnt3, h, w, root, bias2d):
    return pl.pallas_call(
        _pre_body,
        grid=(NBLK,),
        in_specs=[
            pl.BlockSpec((R, BLK, D), lambda i: (0, i, 0)),
            pl.BlockSpec((NSC, BLK, R), lambda i: (0, i, 0)),
            pl.BlockSpec((BLK, D), lambda i: (i, 0)),
            pl.BlockSpec((R, D, D), lambda i: (0, 0, 0)),
            pl.BlockSpec((D, D), lambda i: (0, 0)),
            pl.BlockSpec((1, D), lambda i: (0, 0)),
        ],
        out_specs=[
            pl.BlockSpec((BLK, D), lambda i: (i, 0)),
            pl.BlockSpec((1, 2, D), lambda i: (i, 0, 0)),
        ],
        out_shape=[
            jax.ShapeDtypeStruct((N, D), jnp.float32),
            jax.ShapeDtypeStruct((NBLK, 2, D), jnp.float32),
        ],
    )(s3, cnt3, h, w, root, bias2d)


def _bn_body(pre_ref, st_ref, g_ref, b_ref, h_ref):
    mu = st_ref[:, 0, :].sum(axis=0) * (1.0 / N)
    msq = st_ref[:, 1, :].sum(axis=0) * (1.0 / N)
    var = msq - mu * mu
    scale = lax.rsqrt(var + EPS) * g_ref[0]
    h_ref[...] = jnp.maximum((pre_ref[...] - mu) * scale + b_ref[0], 0.0)


def _bn(pre, st, gamma2d, beta2d):
    return pl.pallas_call(
        _bn_body,
        out_shape=jax.ShapeDtypeStruct((N, D), jnp.float32),
    )(pre, st, gamma2d, beta2d)


def kernel(x, edge_index, edge_type, emb, W1, root1, bias1, gamma1, beta1,
           W2, root2, bias2, gamma2, beta2):
    src = edge_index[0]
    dst = edge_index[1]
    pad = jnp.arange(EXTRA, dtype=jnp.int32)
    src1 = jnp.concatenate([src, pad % N])
    dst1 = jnp.concatenate([dst, jnp.full((EXTRA,), N, jnp.int32)])
    et1 = jnp.concatenate([edge_type, pad % TRASH])
    zeros1 = jnp.zeros((CZR,), jnp.float32)

    seg1, cnt_parts = _count(dst1, et1, zeros1)
    cnt3 = cnt_parts.reshape(NSC, N, R)

    i32 = jnp.int32
    key = edge_type * 2 + (dst >= HALFN).astype(i32)
    sizes = jnp.bincount(key, length=NBK).astype(i32)
    psz = ((sizes + BW - 1) // BW) * BW
    z1 = jnp.zeros((1,), i32)
    poff = jnp.concatenate([z1, jnp.cumsum(psz, dtype=i32)])
    uoff = jnp.concatenate([z1, jnp.cumsum(sizes, dtype=i32)])
    order = jnp.argsort(key, stable=True)
    ks = key[order]
    rank = jnp.arange(E, dtype=i32) - uoff[ks]
    pos = poff[ks] + rank
    local = dst - (key & 1) * HALFN
    pv = local * SB + src
    ar = jnp.arange(CAP, dtype=i32)
    packed = (HALFN + (ar & (BW - 1))) * SB + (ar % 8192)
    packed = packed.at[pos].set(pv[order])
    roff = jnp.zeros((48,), i32).at[: NBK + 1].set(poff // BW)

    h0 = jnp.take(emb, x, axis=0)

    s1 = _segsum(h0, packed, roff)
    pre1, st1 = _pre(s1, cnt3, h0, W1, root1, bias1.reshape(1, D))
    h1 = _bn(pre1, st1, gamma1.reshape(1, D), beta1.reshape(1, D))

    s2 = _segsum(h1, packed, roff)
    pre2, st2 = _pre(s2, cnt3, h1, W2, root2, bias2.reshape(1, D))
    return _bn(pre2, st2, gamma2.reshape(1, D), beta2.reshape(1, D))
